# Initial kernel scaffold; baseline (speedup 1.0000x reference)
import jax
import jax.numpy as jnp
from jax.experimental import pallas as pl


def kernel(x, pos, edge_index, Win, b_in, Wout, b_out, Wlin, Wsrc, Wdst, P1, pb1, P2, pb2, A1, ab1, A2, ab2):
    raise NotImplementedError("write your pallas kernel here")


# speedup vs baseline: 4.3163x; 4.3163x over previous
import functools

import jax
import jax.numpy as jnp
from jax import lax
from jax.experimental import pallas as pl
from jax.experimental.pallas import tpu as pltpu
from jax.experimental.pallas import tpu_sc as plsc

NC = 2
NS = 16
NW = NC * NS
C = 128

F32 = jnp.float32


def _relu(v):
    return jnp.maximum(v, 0.0)


def _dot(a, b):
    return jax.lax.dot_general(a, b, (((1,), (0,)), ((), ())),
                               preferred_element_type=F32)


def _k1_body(x_ref, winT, b_in, wdstT, wsrcT, wlinT, a1T,
             rdst_ref, rsrc_ref, xl_ref):
    h = _relu(_dot(x_ref[...], winT[...]) + b_in[...])
    rdst_ref[...] = _dot(_dot(h, wdstT[...]), a1T[...])
    rsrc_ref[...] = _dot(_dot(h, wsrcT[...]), a1T[...])
    xl_ref[...] = _dot(h, wlinT[...])


def _k1(x_pad, winT, b_in, wdstT, wsrcT, wlinT, a1T, npad):
    bn = 256
    grid = (npad // bn,)
    full = lambda shape: pl.BlockSpec(shape, lambda i: (0, 0))
    return pl.pallas_call(
        _k1_body,
        grid=grid,
        in_specs=[
            pl.BlockSpec((bn, 128), lambda i: (i, 0)),
            full((128, 128)), full((1, 128)),
            full((128, 128)), full((128, 128)), full((128, 128)),
            full((128, 64)),
        ],
        out_specs=[
            pl.BlockSpec((bn, 64), lambda i: (i, 0)),
            pl.BlockSpec((bn, 64), lambda i: (i, 0)),
            pl.BlockSpec((bn, 128), lambda i: (i, 0)),
        ],
        out_shape=[
            jax.ShapeDtypeStruct((npad, 64), F32),
            jax.ShapeDtypeStruct((npad, 64), F32),
            jax.ShapeDtypeStruct((npad, 128), F32),
        ],
    )(x_pad, winT, b_in, wdstT, wsrcT, wlinT, a1T)


def _k2_body(ep, src_hbm, dst_hbm, pos_hbm, rdst_hbm, rsrc_hbm, xl_hbm,
             opd, ops, ord_, ors, oxl,
             idx_s, idx_d, bpd, bps, brd, brs, bxl, sem):
    c = lax.axis_index("c")
    s = lax.axis_index("s")
    wid = s * NC + c
    ew = ep // NW

    def chunk(k, _):
        base = wid * ew + k * C
        pltpu.sync_copy(src_hbm.at[pl.ds(base, C)], idx_s)
        pltpu.sync_copy(dst_hbm.at[pl.ds(base, C)], idx_d)
        g1 = pltpu.async_copy(pos_hbm.at[idx_d], bpd, sem)
        g2 = pltpu.async_copy(pos_hbm.at[idx_s], bps, sem)
        g3 = pltpu.async_copy(rdst_hbm.at[idx_d], brd, sem)
        g4 = pltpu.async_copy(rsrc_hbm.at[idx_s], brs, sem)
        g5 = pltpu.async_copy(xl_hbm.at[idx_s], bxl, sem)
        g1.wait(); g2.wait(); g3.wait(); g4.wait(); g5.wait()
        pltpu.sync_copy(bpd, opd.at[pl.ds(base, C)])
        pltpu.sync_copy(bps, ops.at[pl.ds(base, C)])
        pltpu.sync_copy(brd, ord_.at[pl.ds(base, C)])
        pltpu.sync_copy(brs, ors.at[pl.ds(base, C)])
        pltpu.sync_copy(bxl, oxl.at[pl.ds(base, C)])
        return 0

    lax.fori_loop(0, ew // C, chunk, 0)


def _k2(src, dst, pos16, rdst, rsrc, xl, ep):
    mesh = plsc.VectorSubcoreMesh(core_axis_name="c", subcore_axis_name="s")
    kern = functools.partial(
        pl.kernel,
        compiler_params=pltpu.CompilerParams(use_tc_tiling_on_sc=False),
        out_type=[
            jax.ShapeDtypeStruct((ep, 16), F32),
            jax.ShapeDtypeStruct((ep, 16), F32),
            jax.ShapeDtypeStruct((ep, 64), F32),
            jax.ShapeDtypeStruct((ep, 64), F32),
            jax.ShapeDtypeStruct((ep, 128), F32),
        ],
        mesh=mesh,
        scratch_types=[
            pltpu.VMEM((C,), jnp.int32),
            pltpu.VMEM((C,), jnp.int32),
            pltpu.VMEM((C, 16), F32),
            pltpu.VMEM((C, 16), F32),
            pltpu.VMEM((C, 64), F32),
            pltpu.VMEM((C, 64), F32),
            pltpu.VMEM((C, 128), F32),
            pltpu.SemaphoreType.DMA,
        ],
    )(functools.partial(_k2_body, ep))
    return kern(src, dst, pos16, rdst, rsrc, xl)


def _k3_body(pd_ref, ps_ref, rd_ref, rs_ref, xl_ref,
             p1t, pb1, p2t, pb2, a1t, ab1, a2t, ab2,
             ex_ref, pp_ref):
    pd = pd_ref[...] - ps_ref[...]
    d1 = _relu(_dot(pd, p1t[...]) + pb1[...])
    delta = _relu(_dot(d1, p2t[...]) + pb2[...])
    z1 = _relu(_dot(delta, a1t[...]) + rd_ref[...] - rs_ref[...] + ab1[...])
    alpha = _relu(_dot(z1, a2t[...]) + ab2[...])
    ex = jnp.exp(alpha)
    ex_ref[...] = ex
    pp_ref[...] = ex * (xl_ref[...] + delta)


def _k3(pd, ps, rd, rs, xlg, p1t, pb1, p2t, pb2, a1t, ab1, a2t, ab2, ep):
    be = 512
    grid = (ep // be,)
    full = lambda shape: pl.BlockSpec(shape, lambda i: (0, 0))
    row = lambda w: pl.BlockSpec((be, w), lambda i: (i, 0))
    return pl.pallas_call(
        _k3_body,
        grid=grid,
        in_specs=[
            row(16), row(16), row(64), row(64), row(128),
            full((16, 64)), full((1, 64)), full((64, 128)), full((1, 128)),
            full((128, 64)), full((1, 64)), full((64, 128)), full((1, 128)),
        ],
        out_specs=[row(128), row(128)],
        out_shape=[
            jax.ShapeDtypeStruct((ep, 128), F32),
            jax.ShapeDtypeStruct((ep, 128), F32),
        ],
    )(pd, ps, rd, rs, xlg, p1t, pb1, p2t, pb2, a1t, ab1, a2t, ab2)


def _k4_body(ep, npad, dst_hbm, ex_hbm, pp_hbm, denom_hbm, num_hbm,
             acc, idxb, rowb, zbuf, sem):
    c = lax.axis_index("c")
    s = lax.axis_index("s")
    rows_per_tile = npad // NS
    r0 = s * rows_per_tile

    def zrow(j, _):
        for t in range(8):
            zbuf[j, pl.ds(t * 16, 16)] = jnp.zeros((16,), F32)
        return 0

    lax.fori_loop(0, C, zrow, 0)
    for t in range(rows_per_tile // C):
        pltpu.sync_copy(zbuf, acc.at[pl.ds(r0 + t * C, C)])
    plsc.subcore_barrier()

    et = ep // NS

    def chunk_from(src_arr):
        def chunk(k, _):
            base = s * et + k * C
            pltpu.sync_copy(dst_hbm.at[pl.ds(base, C)], idxb)
            pltpu.sync_copy(src_arr.at[pl.ds(base, C)], rowb)
            pltpu.sync_copy(rowb, acc.at[idxb], add=True)
            return 0
        lax.fori_loop(0, et // C, chunk, 0)

    @pl.when(c == 0)
    def _():
        chunk_from(ex_hbm)

    @pl.when(c == 1)
    def _():
        chunk_from(pp_hbm)

    plsc.subcore_barrier()
    for t in range(rows_per_tile // C):
        rows = pl.ds(r0 + t * C, C)

        @pl.when(c == 0)
        def _(rows=rows):
            pltpu.sync_copy(acc.at[rows], denom_hbm.at[rows])

        @pl.when(c == 1)
        def _(rows=rows):
            pltpu.sync_copy(acc.at[rows], num_hbm.at[rows])


def _k4(dst, ex, pp, ep, npad):
    mesh = plsc.VectorSubcoreMesh(core_axis_name="c", subcore_axis_name="s")
    kern = functools.partial(
        pl.kernel,
        out_type=[
            jax.ShapeDtypeStruct((npad, 128), F32),
            jax.ShapeDtypeStruct((npad, 128), F32),
        ],
        mesh=mesh,
        scratch_types=[
            pltpu.VMEM_SHARED((npad, 128), F32),
            pltpu.VMEM((C,), jnp.int32),
            pltpu.VMEM((C, 128), F32),
            pltpu.VMEM((C, 128), F32),
            pltpu.SemaphoreType.DMA,
        ],
    )(functools.partial(_k4_body, ep, npad))
    return kern(dst, ex, pp)


def _k5_body(num_ref, den_ref, woutT, b_out, y_ref):
    out = num_ref[...] / (den_ref[...] + 1e-16)
    y_ref[...] = _relu(_dot(out, woutT[...]) + b_out[...])


def _k5(num, den, woutT, b_out, npad):
    bn = 256
    grid = (npad // bn,)
    full = lambda shape: pl.BlockSpec(shape, lambda i: (0, 0))
    return pl.pallas_call(
        _k5_body,
        grid=grid,
        in_specs=[
            pl.BlockSpec((bn, 128), lambda i: (i, 0)),
            pl.BlockSpec((bn, 128), lambda i: (i, 0)),
            full((128, 128)), full((1, 128)),
        ],
        out_specs=pl.BlockSpec((bn, 128), lambda i: (i, 0)),
        out_shape=jax.ShapeDtypeStruct((npad, 128), F32),
    )(num, den, woutT, b_out)


def kernel(x, pos, edge_index, Win, b_in, Wout, b_out, Wlin, Wsrc, Wdst,
           P1, pb1, P2, pb2, A1, ab1, A2, ab2):
    n, d = x.shape
    e = edge_index.shape[1]
    npad = ((n + 255) // 256) * 256
    ereal = e + n
    ep = ((ereal + NW * C - 1) // (NW * C)) * (NW * C)

    x_pad = jnp.pad(x, ((0, npad - n), (0, 0)))
    pos16 = jnp.pad(pos, ((0, npad - n), (0, 16 - pos.shape[1])))
    loop = jnp.arange(n, dtype=edge_index.dtype)
    pad_e = ep - ereal
    src = jnp.concatenate([edge_index[0], loop,
                           jnp.zeros((pad_e,), edge_index.dtype)])
    dst = jnp.concatenate([edge_index[1], loop,
                           jnp.full((pad_e,), n, edge_index.dtype)])

    winT = Win.T
    wdstT = Wdst.T
    wsrcT = Wsrc.T
    wlinT = Wlin.T
    woutT = Wout.T
    a1T = A1.T
    p1t = jnp.pad(P1.T, ((0, 16 - P1.shape[1]), (0, 0)))
    p2t = P2.T
    a2t = A2.T
    b_in2 = b_in.reshape(1, -1)
    b_out2 = b_out.reshape(1, -1)
    pb1_2 = pb1.reshape(1, -1)
    pb2_2 = pb2.reshape(1, -1)
    ab1_2 = ab1.reshape(1, -1)
    ab2_2 = ab2.reshape(1, -1)

    rdst, rsrc, xl = _k1(x_pad, winT, b_in2, wdstT, wsrcT, wlinT, a1T, npad)
    pd, ps, rdg, rsg, xlg = _k2(src, dst, pos16, rdst, rsrc, xl, ep)
    ex, pp = _k3(pd, ps, rdg, rsg, xlg,
                 p1t, pb1_2, p2t, pb2_2, a1T, ab1_2, a2t, ab2_2, ep)
    den, num = _k4(dst, ex, pp, ep, npad)
    y = _k5(num, den, woutT, b_out2, npad)
    return y[:n]


# speedup vs baseline: 4.8476x; 1.1231x over previous
import functools

import jax
import jax.numpy as jnp
from jax import lax
from jax.experimental import pallas as pl
from jax.experimental.pallas import tpu as pltpu
from jax.experimental.pallas import tpu_sc as plsc

NC = 2
NS = 16
NW = NC * NS
C = 128

F32 = jnp.float32


def _relu(v):
    return jnp.maximum(v, 0.0)


def _dot(a, b):
    return jax.lax.dot_general(a, b, (((1,), (0,)), ((), ())),
                               preferred_element_type=F32)


def _k1_body(x_ref, pos_ref, winT, b_in, wdstT, wsrcT, wlinT, a1T, p1t,
             td_ref, ts_ref, xl_ref):
    h = _relu(_dot(x_ref[...], winT[...]) + b_in[...])
    q = _dot(pos_ref[...], p1t[...])
    td_ref[...] = jnp.concatenate(
        [_dot(_dot(h, wdstT[...]), a1T[...]), q], axis=1)
    ts_ref[...] = jnp.concatenate(
        [_dot(_dot(h, wsrcT[...]), a1T[...]), q], axis=1)
    xl_ref[...] = _dot(h, wlinT[...])


def _k1(x_pad, pos16, winT, b_in, wdstT, wsrcT, wlinT, a1T, p1t, npad):
    bn = 256
    grid = (npad // bn,)
    full = lambda shape: pl.BlockSpec(shape, lambda i: (0, 0))
    return pl.pallas_call(
        _k1_body,
        grid=grid,
        in_specs=[
            pl.BlockSpec((bn, 128), lambda i: (i, 0)),
            pl.BlockSpec((bn, 128), lambda i: (i, 0)),
            full((128, 128)), full((1, 128)),
            full((128, 128)), full((128, 128)), full((128, 128)),
            full((128, 64)), full((128, 64)),
        ],
        out_specs=[
            pl.BlockSpec((bn, 128), lambda i: (i, 0)),
            pl.BlockSpec((bn, 128), lambda i: (i, 0)),
            pl.BlockSpec((bn, 128), lambda i: (i, 0)),
        ],
        out_shape=[
            jax.ShapeDtypeStruct((npad, 128), F32),
            jax.ShapeDtypeStruct((npad, 128), F32),
            jax.ShapeDtypeStruct((npad, 128), F32),
        ],
    )(x_pad, pos16, winT, b_in, wdstT, wsrcT, wlinT, a1T, p1t)


def _k2_body(ep, src_hbm, dst_hbm, td_hbm, ts_hbm, xl_hbm,
             od, os_, ox,
             idxd0, idxs0, bd0, bs0, bx0,
             idxd1, idxs1, bd1, bs1, bx1,
             gsem0, gsem1, ssem0, ssem1):
    c = lax.axis_index("c")
    s = lax.axis_index("s")
    wid = s * NC + c
    ew = ep // NW
    nk = ew // C
    base0 = wid * ew

    bufs = ((idxd0, idxs0, bd0, bs0, bx0, gsem0, ssem0),
            (idxd1, idxs1, bd1, bs1, bx1, gsem1, ssem1))

    def fire_gather(k, p):
        idxd, idxs, bd, bs, bx, gsem, _ = bufs[p]
        pltpu.sync_copy(dst_hbm.at[pl.ds(base0 + k * C, C)], idxd)
        pltpu.sync_copy(src_hbm.at[pl.ds(base0 + k * C, C)], idxs)
        pltpu.async_copy(td_hbm.at[idxd], bd, gsem)
        pltpu.async_copy(ts_hbm.at[idxs], bs, gsem)
        pltpu.async_copy(xl_hbm.at[idxs], bx, gsem)

    def wait_gather(p):
        idxd, idxs, bd, bs, bx, gsem, _ = bufs[p]
        pltpu.make_async_copy(td_hbm.at[idxd], bd, gsem).wait()
        pltpu.make_async_copy(ts_hbm.at[idxs], bs, gsem).wait()
        pltpu.make_async_copy(xl_hbm.at[idxs], bx, gsem).wait()

    def fire_store(k, p):
        _, _, bd, bs, bx, _, ssem = bufs[p]
        sl = pl.ds(base0 + k * C, C)
        pltpu.async_copy(bd, od.at[sl], ssem)
        pltpu.async_copy(bs, os_.at[sl], ssem)
        pltpu.async_copy(bx, ox.at[sl], ssem)

    def wait_store(k, p):
        _, _, bd, bs, bx, _, ssem = bufs[p]
        sl = pl.ds(base0 + k * C, C)
        pltpu.make_async_copy(bd, od.at[sl], ssem).wait()
        pltpu.make_async_copy(bs, os_.at[sl], ssem).wait()
        pltpu.make_async_copy(bx, ox.at[sl], ssem).wait()

    def body(k, _):
        fire_gather(k, 0)
        wait_gather(0)
        fire_store(k, 0)
        wait_store(k, 0)
        return 0

    lax.fori_loop(0, nk, body, 0)


def _k2(src, dst, td, ts, xl, ep):
    mesh = plsc.VectorSubcoreMesh(core_axis_name="c", subcore_axis_name="s")
    buf = lambda: [pltpu.VMEM((C,), jnp.int32), pltpu.VMEM((C,), jnp.int32),
                   pltpu.VMEM((C, 128), F32), pltpu.VMEM((C, 128), F32),
                   pltpu.VMEM((C, 128), F32)]
    kern = functools.partial(
        pl.kernel,
        compiler_params=pltpu.CompilerParams(use_tc_tiling_on_sc=False),
        out_type=[
            jax.ShapeDtypeStruct((ep, 128), F32),
            jax.ShapeDtypeStruct((ep, 128), F32),
            jax.ShapeDtypeStruct((ep, 128), F32),
        ],
        mesh=mesh,
        scratch_types=buf() + buf() + [pltpu.SemaphoreType.DMA] * 4,
    )(functools.partial(_k2_body, ep))
    return kern(src, dst, td, ts, xl)


def _k3_body(d_ref, s_ref, x_ref,
             e1sel, e2sel, pb1, p2t, pb2, a1t, ab1, a2t, ab2,
             ex_ref, pp_ref):
    diff = d_ref[...] - s_ref[...]
    d1 = _relu(_dot(diff, e2sel[...]) + pb1[...])
    delta = _relu(_dot(d1, p2t[...]) + pb2[...])
    z1 = _relu(_dot(delta, a1t[...]) + _dot(diff, e1sel[...])
               + ab1[...])
    alpha = _relu(_dot(z1, a2t[...]) + ab2[...])
    ex = jnp.exp(alpha)
    ex_ref[...] = ex
    pp_ref[...] = ex * (x_ref[...] + delta)


def _k3(dg, sg, xg, pb1, p2t, pb2, a1t, ab1, a2t, ab2, ep):
    be = 1024
    grid = (ep // be,)
    eye = jnp.eye(128, dtype=F32)
    e1sel = eye[:, :64]
    e2sel = eye[:, 64:]
    full = lambda shape: pl.BlockSpec(shape, lambda i: (0, 0))
    row = pl.BlockSpec((be, 128), lambda i: (i, 0))
    return pl.pallas_call(
        _k3_body,
        grid=grid,
        in_specs=[
            row, row, row,
            full((128, 64)), full((128, 64)),
            full((1, 64)), full((64, 128)), full((1, 128)),
            full((128, 64)), full((1, 64)), full((64, 128)), full((1, 128)),
        ],
        out_specs=[row, row],
        out_shape=[
            jax.ShapeDtypeStruct((ep, 128), F32),
            jax.ShapeDtypeStruct((ep, 128), F32),
        ],
    )(dg, sg, xg, e1sel, e2sel, pb1, p2t, pb2, a1t, ab1, a2t, ab2)


def _k4_body(ep, npad, dst_hbm, ex_hbm, pp_hbm, denom_hbm, num_hbm,
             acc, idxb0, rowb0, idxb1, rowb1, fsem0, fsem1):
    c = lax.axis_index("c")
    s = lax.axis_index("s")
    rows_per_tile = npad // NS
    r0 = s * rows_per_tile

    def zrow(j, _):
        for t in range(8):
            rowb0[j, pl.ds(t * 16, 16)] = jnp.zeros((16,), F32)
        return 0

    lax.fori_loop(0, C, zrow, 0)
    for t in range(rows_per_tile // C):
        pltpu.sync_copy(rowb0, acc.at[pl.ds(r0 + t * C, C)])
    plsc.subcore_barrier()

    et = ep // NS
    nk = et // C
    ng = nk // 2
    base0 = s * et
    bufs = ((idxb0, rowb0, fsem0), (idxb1, rowb1, fsem1))

    def chunk_from(src_arr):
        def fire(k, p):
            idxb, rowb, fsem = bufs[p]
            pltpu.sync_copy(dst_hbm.at[pl.ds(base0 + k * C, C)], idxb)
            pltpu.async_copy(src_arr.at[pl.ds(base0 + k * C, C)], rowb, fsem)

        def scat(k, p):
            idxb, rowb, fsem = bufs[p]
            pltpu.make_async_copy(
                src_arr.at[pl.ds(base0 + k * C, C)], rowb, fsem).wait()
            pltpu.sync_copy(rowb, acc.at[idxb], add=True)

        def body(k, _):
            fire(k, 0)
            scat(k, 0)
            return 0

        lax.fori_loop(0, nk, body, 0)

    @pl.when(c == 0)
    def _():
        chunk_from(ex_hbm)

    @pl.when(c == 1)
    def _():
        chunk_from(pp_hbm)

    plsc.subcore_barrier()
    for t in range(rows_per_tile // C):
        rows = pl.ds(r0 + t * C, C)

        @pl.when(c == 0)
        def _(rows=rows):
            pltpu.sync_copy(acc.at[rows], denom_hbm.at[rows])

        @pl.when(c == 1)
        def _(rows=rows):
            pltpu.sync_copy(acc.at[rows], num_hbm.at[rows])


def _k4(dst, ex, pp, ep, npad):
    mesh = plsc.VectorSubcoreMesh(core_axis_name="c", subcore_axis_name="s")
    kern = functools.partial(
        pl.kernel,
        out_type=[
            jax.ShapeDtypeStruct((npad, 128), F32),
            jax.ShapeDtypeStruct((npad, 128), F32),
        ],
        mesh=mesh,
        scratch_types=[
            pltpu.VMEM_SHARED((npad, 128), F32),
            pltpu.VMEM((C,), jnp.int32),
            pltpu.VMEM((C, 128), F32),
            pltpu.VMEM((C,), jnp.int32),
            pltpu.VMEM((C, 128), F32),
            pltpu.SemaphoreType.DMA,
            pltpu.SemaphoreType.DMA,
        ],
    )(functools.partial(_k4_body, ep, npad))
    return kern(dst, ex, pp)


def _k5_body(num_ref, den_ref, woutT, b_out, y_ref):
    out = num_ref[...] / (den_ref[...] + 1e-16)
    y_ref[...] = _relu(_dot(out, woutT[...]) + b_out[...])


def _k5(num, den, woutT, b_out, npad):
    bn = 256
    grid = (npad // bn,)
    full = lambda shape: pl.BlockSpec(shape, lambda i: (0, 0))
    return pl.pallas_call(
        _k5_body,
        grid=grid,
        in_specs=[
            pl.BlockSpec((bn, 128), lambda i: (i, 0)),
            pl.BlockSpec((bn, 128), lambda i: (i, 0)),
            full((128, 128)), full((1, 128)),
        ],
        out_specs=pl.BlockSpec((bn, 128), lambda i: (i, 0)),
        out_shape=jax.ShapeDtypeStruct((npad, 128), F32),
    )(num, den, woutT, b_out)


def kernel(x, pos, edge_index, Win, b_in, Wout, b_out, Wlin, Wsrc, Wdst,
           P1, pb1, P2, pb2, A1, ab1, A2, ab2):
    n, d = x.shape
    e = edge_index.shape[1]
    npad = ((n + 255) // 256) * 256
    ereal = e + n
    ep = ((ereal + 2 * NW * C - 1) // (2 * NW * C)) * (2 * NW * C)

    x_pad = jnp.pad(x, ((0, npad - n), (0, 0)))
    pos16 = jnp.pad(pos, ((0, npad - n), (0, 128 - pos.shape[1])))
    loop = jnp.arange(n, dtype=edge_index.dtype)
    pad_e = ep - ereal
    src = jnp.concatenate([edge_index[0], loop,
                           jnp.zeros((pad_e,), edge_index.dtype)])
    dst = jnp.concatenate([edge_index[1], loop,
                           jnp.full((pad_e,), n, edge_index.dtype)])

    winT = Win.T
    wdstT = Wdst.T
    wsrcT = Wsrc.T
    wlinT = Wlin.T
    woutT = Wout.T
    a1T = A1.T
    p1t = jnp.pad(P1.T, ((0, 128 - P1.shape[1]), (0, 0)))
    p2t = P2.T
    a2t = A2.T
    b_in2 = b_in.reshape(1, -1)
    b_out2 = b_out.reshape(1, -1)
    pb1_2 = pb1.reshape(1, -1)
    pb2_2 = pb2.reshape(1, -1)
    ab1_2 = ab1.reshape(1, -1)
    ab2_2 = ab2.reshape(1, -1)

    td, ts, xl = _k1(x_pad, pos16, winT, b_in2, wdstT, wsrcT, wlinT,
                     a1T, p1t, npad)
    dg, sg, xg = _k2(src, dst, td, ts, xl, ep)
    ex, pp = _k3(dg, sg, xg, pb1_2, p2t, pb2_2, a1T, ab1_2, a2t, ab2_2, ep)
    den, num = _k4(dst, ex, pp, ep, npad)
    y = _k5(num, den, woutT, b_out2, npad)
    return y[:n]


# speedup vs baseline: 6.2679x; 1.2930x over previous
import functools

import jax
import jax.numpy as jnp
from jax import lax
from jax.experimental import pallas as pl
from jax.experimental.pallas import tpu as pltpu
from jax.experimental.pallas import tpu_sc as plsc

NC = 2
NS = 16
NW = NC * NS
C = 128

F32 = jnp.float32


def _relu(v):
    return jnp.maximum(v, 0.0)


def _dot(a, b):
    return jax.lax.dot_general(a, b, (((1,), (0,)), ((), ())),
                               preferred_element_type=F32)


def _k1_body(x_ref, pos_ref, winT, b_in, wdstT, wsrcT, wlinT, a1T, p1t,
             td_ref, ts_ref, xl_ref):
    h = _relu(_dot(x_ref[...], winT[...]) + b_in[...])
    q = _dot(pos_ref[...], p1t[...])
    td_ref[...] = jnp.concatenate(
        [_dot(_dot(h, wdstT[...]), a1T[...]), q], axis=1)
    ts_ref[...] = jnp.concatenate(
        [_dot(_dot(h, wsrcT[...]), a1T[...]), q], axis=1)
    xl_ref[...] = _dot(h, wlinT[...])


def _k1(x_pad, pos16, winT, b_in, wdstT, wsrcT, wlinT, a1T, p1t, npad):
    bn = 256
    grid = (npad // bn,)
    full = lambda shape: pl.BlockSpec(shape, lambda i: (0, 0))
    return pl.pallas_call(
        _k1_body,
        grid=grid,
        in_specs=[
            pl.BlockSpec((bn, 128), lambda i: (i, 0)),
            pl.BlockSpec((bn, 128), lambda i: (i, 0)),
            full((128, 128)), full((1, 128)),
            full((128, 128)), full((128, 128)), full((128, 128)),
            full((128, 64)), full((128, 64)),
        ],
        out_specs=[
            pl.BlockSpec((bn, 128), lambda i: (i, 0)),
            pl.BlockSpec((bn, 128), lambda i: (i, 0)),
            pl.BlockSpec((bn, 128), lambda i: (i, 0)),
        ],
        out_shape=[
            jax.ShapeDtypeStruct((npad, 128), F32),
            jax.ShapeDtypeStruct((npad, 128), F32),
            jax.ShapeDtypeStruct((npad, 128), F32),
        ],
    )(x_pad, pos16, winT, b_in, wdstT, wsrcT, wlinT, a1T, p1t)


def _k2_body(ep, src_hbm, dst_hbm, td_hbm, ts_hbm, xl_hbm,
             od, os_, ox,
             idxd0, idxs0, bd0, bs0, bx0,
             idxd1, idxs1, bd1, bs1, bx1,
             gsem0, gsem1, ssem0, ssem1):
    c = lax.axis_index("c")
    s = lax.axis_index("s")
    wid = s * NC + c
    ew = ep // NW
    nk = ew // C
    base0 = wid * ew

    bufs = ((idxd0, idxs0, bd0, bs0, bx0, gsem0, ssem0),
            (idxd1, idxs1, bd1, bs1, bx1, gsem1, ssem1))

    def fire_gather(k, p):
        idxd, idxs, bd, bs, bx, gsem, _ = bufs[p]
        pltpu.sync_copy(dst_hbm.at[pl.ds(base0 + k * C, C)], idxd)
        pltpu.sync_copy(src_hbm.at[pl.ds(base0 + k * C, C)], idxs)
        pltpu.async_copy(td_hbm.at[idxd], bd, gsem)
        pltpu.async_copy(ts_hbm.at[idxs], bs, gsem)
        pltpu.async_copy(xl_hbm.at[idxs], bx, gsem)

    def wait_gather(p):
        idxd, idxs, bd, bs, bx, gsem, _ = bufs[p]
        pltpu.make_async_copy(td_hbm.at[idxd], bd, gsem).wait()
        pltpu.make_async_copy(ts_hbm.at[idxs], bs, gsem).wait()
        pltpu.make_async_copy(xl_hbm.at[idxs], bx, gsem).wait()

    def fire_store(k, p):
        _, _, bd, bs, bx, _, ssem = bufs[p]
        sl = pl.ds(base0 + k * C, C)
        pltpu.async_copy(bd, od.at[sl], ssem)
        pltpu.async_copy(bs, os_.at[sl], ssem)
        pltpu.async_copy(bx, ox.at[sl], ssem)

    def wait_store(k, p):
        _, _, bd, bs, bx, _, ssem = bufs[p]
        sl = pl.ds(base0 + k * C, C)
        pltpu.make_async_copy(bd, od.at[sl], ssem).wait()
        pltpu.make_async_copy(bs, os_.at[sl], ssem).wait()
        pltpu.make_async_copy(bx, ox.at[sl], ssem).wait()

    fire_gather(0, 0)
    ng = nk // 2

    def body(g, _):
        k0 = 2 * g
        k1 = k0 + 1

        @pl.when(g > 0)
        def _():
            wait_store(k0 - 1, 1)

        fire_gather(k1, 1)
        wait_gather(0)
        fire_store(k0, 0)
        wait_store(k0, 0)

        @pl.when(g < ng - 1)
        def _():
            fire_gather(k0 + 2, 0)

        wait_gather(1)
        fire_store(k1, 1)
        return 0

    lax.fori_loop(0, ng, body, 0)
    wait_store(nk - 1, 1)


def _k2(src, dst, td, ts, xl, ep):
    mesh = plsc.VectorSubcoreMesh(core_axis_name="c", subcore_axis_name="s")
    buf = lambda: [pltpu.VMEM((C,), jnp.int32), pltpu.VMEM((C,), jnp.int32),
                   pltpu.VMEM((C, 128), F32), pltpu.VMEM((C, 128), F32),
                   pltpu.VMEM((C, 128), F32)]
    kern = functools.partial(
        pl.kernel,
        compiler_params=pltpu.CompilerParams(use_tc_tiling_on_sc=False),
        out_type=[
            jax.ShapeDtypeStruct((ep, 128), F32),
            jax.ShapeDtypeStruct((ep, 128), F32),
            jax.ShapeDtypeStruct((ep, 128), F32),
        ],
        mesh=mesh,
        scratch_types=buf() + buf() + [pltpu.SemaphoreType.DMA] * 4,
    )(functools.partial(_k2_body, ep))
    return kern(src, dst, td, ts, xl)


def _k3_body(d_ref, s_ref, x_ref,
             e1sel, e2sel, pb1, p2t, pb2, a1t, ab1, a2t, ab2,
             ex_ref, pp_ref):
    diff = d_ref[...] - s_ref[...]
    d1 = _relu(_dot(diff, e2sel[...]) + pb1[...])
    delta = _relu(_dot(d1, p2t[...]) + pb2[...])
    z1 = _relu(_dot(delta, a1t[...]) + _dot(diff, e1sel[...])
               + ab1[...])
    alpha = _relu(_dot(z1, a2t[...]) + ab2[...])
    ex = jnp.exp(alpha)
    ex_ref[...] = ex
    pp_ref[...] = ex * (x_ref[...] + delta)


def _k3(dg, sg, xg, pb1, p2t, pb2, a1t, ab1, a2t, ab2, ep):
    be = 1024
    grid = (ep // be,)
    eye = jnp.eye(128, dtype=F32)
    e1sel = eye[:, :64]
    e2sel = eye[:, 64:]
    full = lambda shape: pl.BlockSpec(shape, lambda i: (0, 0))
    row = pl.BlockSpec((be, 128), lambda i: (i, 0))
    return pl.pallas_call(
        _k3_body,
        grid=grid,
        in_specs=[
            row, row, row,
            full((128, 64)), full((128, 64)),
            full((1, 64)), full((64, 128)), full((1, 128)),
            full((128, 64)), full((1, 64)), full((64, 128)), full((1, 128)),
        ],
        out_specs=[row, row],
        out_shape=[
            jax.ShapeDtypeStruct((ep, 128), F32),
            jax.ShapeDtypeStruct((ep, 128), F32),
        ],
    )(dg, sg, xg, e1sel, e2sel, pb1, p2t, pb2, a1t, ab1, a2t, ab2)


def _k4_body(ep, npad, dst_hbm, ex_hbm, pp_hbm, denom_hbm, num_hbm,
             acc, idxb0, rowb0, idxb1, rowb1, fsem0, fsem1):
    c = lax.axis_index("c")
    s = lax.axis_index("s")
    rows_per_tile = npad // NS
    r0 = s * rows_per_tile

    def zrow(j, _):
        for t in range(8):
            rowb0[j, pl.ds(t * 16, 16)] = jnp.zeros((16,), F32)
        return 0

    lax.fori_loop(0, C, zrow, 0)
    for t in range(rows_per_tile // C):
        pltpu.sync_copy(rowb0, acc.at[pl.ds(r0 + t * C, C)])
    plsc.subcore_barrier()

    et = ep // NS
    nk = et // C
    ng = nk // 2
    base0 = s * et
    bufs = ((idxb0, rowb0, fsem0), (idxb1, rowb1, fsem1))

    def chunk_from(src_arr):
        def fire(k, p):
            idxb, rowb, fsem = bufs[p]
            pltpu.sync_copy(dst_hbm.at[pl.ds(base0 + k * C, C)], idxb)
            pltpu.async_copy(src_arr.at[pl.ds(base0 + k * C, C)], rowb, fsem)

        def scat(k, p):
            idxb, rowb, fsem = bufs[p]
            pltpu.make_async_copy(
                src_arr.at[pl.ds(base0 + k * C, C)], rowb, fsem).wait()
            pltpu.sync_copy(rowb, acc.at[idxb], add=True)

        fire(0, 0)

        def body(g, _):
            k0 = 2 * g
            k1 = k0 + 1
            fire(k1, 1)
            scat(k0, 0)

            @pl.when(g < ng - 1)
            def _():
                fire(k0 + 2, 0)

            scat(k1, 1)
            return 0

        lax.fori_loop(0, ng, body, 0)

    @pl.when(c == 0)
    def _():
        chunk_from(ex_hbm)

    @pl.when(c == 1)
    def _():
        chunk_from(pp_hbm)

    plsc.subcore_barrier()
    for t in range(rows_per_tile // C):
        rows = pl.ds(r0 + t * C, C)

        @pl.when(c == 0)
        def _(rows=rows):
            pltpu.sync_copy(acc.at[rows], denom_hbm.at[rows])

        @pl.when(c == 1)
        def _(rows=rows):
            pltpu.sync_copy(acc.at[rows], num_hbm.at[rows])


def _k4(dst, ex, pp, ep, npad):
    mesh = plsc.VectorSubcoreMesh(core_axis_name="c", subcore_axis_name="s")
    kern = functools.partial(
        pl.kernel,
        out_type=[
            jax.ShapeDtypeStruct((npad, 128), F32),
            jax.ShapeDtypeStruct((npad, 128), F32),
        ],
        mesh=mesh,
        scratch_types=[
            pltpu.VMEM_SHARED((npad, 128), F32),
            pltpu.VMEM((C,), jnp.int32),
            pltpu.VMEM((C, 128), F32),
            pltpu.VMEM((C,), jnp.int32),
            pltpu.VMEM((C, 128), F32),
            pltpu.SemaphoreType.DMA,
            pltpu.SemaphoreType.DMA,
        ],
    )(functools.partial(_k4_body, ep, npad))
    return kern(dst, ex, pp)


def _k5_body(num_ref, den_ref, woutT, b_out, y_ref):
    out = num_ref[...] / (den_ref[...] + 1e-16)
    y_ref[...] = _relu(_dot(out, woutT[...]) + b_out[...])


def _k5(num, den, woutT, b_out, npad):
    bn = 256
    grid = (npad // bn,)
    full = lambda shape: pl.BlockSpec(shape, lambda i: (0, 0))
    return pl.pallas_call(
        _k5_body,
        grid=grid,
        in_specs=[
            pl.BlockSpec((bn, 128), lambda i: (i, 0)),
            pl.BlockSpec((bn, 128), lambda i: (i, 0)),
            full((128, 128)), full((1, 128)),
        ],
        out_specs=pl.BlockSpec((bn, 128), lambda i: (i, 0)),
        out_shape=jax.ShapeDtypeStruct((npad, 128), F32),
    )(num, den, woutT, b_out)


def kernel(x, pos, edge_index, Win, b_in, Wout, b_out, Wlin, Wsrc, Wdst,
           P1, pb1, P2, pb2, A1, ab1, A2, ab2):
    n, d = x.shape
    e = edge_index.shape[1]
    npad = ((n + 255) // 256) * 256
    ereal = e + n
    ep = ((ereal + 2 * NW * C - 1) // (2 * NW * C)) * (2 * NW * C)

    x_pad = jnp.pad(x, ((0, npad - n), (0, 0)))
    pos16 = jnp.pad(pos, ((0, npad - n), (0, 128 - pos.shape[1])))
    loop = jnp.arange(n, dtype=edge_index.dtype)
    pad_e = ep - ereal
    src = jnp.concatenate([edge_index[0], loop,
                           jnp.zeros((pad_e,), edge_index.dtype)])
    dst = jnp.concatenate([edge_index[1], loop,
                           jnp.full((pad_e,), n, edge_index.dtype)])

    winT = Win.T
    wdstT = Wdst.T
    wsrcT = Wsrc.T
    wlinT = Wlin.T
    woutT = Wout.T
    a1T = A1.T
    p1t = jnp.pad(P1.T, ((0, 128 - P1.shape[1]), (0, 0)))
    p2t = P2.T
    a2t = A2.T
    b_in2 = b_in.reshape(1, -1)
    b_out2 = b_out.reshape(1, -1)
    pb1_2 = pb1.reshape(1, -1)
    pb2_2 = pb2.reshape(1, -1)
    ab1_2 = ab1.reshape(1, -1)
    ab2_2 = ab2.reshape(1, -1)

    td, ts, xl = _k1(x_pad, pos16, winT, b_in2, wdstT, wsrcT, wlinT,
                     a1T, p1t, npad)
    dg, sg, xg = _k2(src, dst, td, ts, xl, ep)
    ex, pp = _k3(dg, sg, xg, pb1_2, p2t, pb2_2, a1T, ab1_2, a2t, ab2_2, ep)
    den, num = _k4(dst, ex, pp, ep, npad)
    y = _k5(num, den, woutT, b_out2, npad)
    return y[:n]


# speedup vs baseline: 6.5673x; 1.0478x over previous
import functools

import jax
import jax.numpy as jnp
from jax import lax
from jax.experimental import pallas as pl
from jax.experimental.pallas import tpu as pltpu
from jax.experimental.pallas import tpu_sc as plsc

NC = 2
NS = 16
NW = NC * NS
C = 128

F32 = jnp.float32


def _relu(v):
    return jnp.maximum(v, 0.0)


def _dot(a, b):
    return jax.lax.dot_general(a, b, (((1,), (0,)), ((), ())),
                               preferred_element_type=F32)


def _k1_body(x_ref, pos_ref, winT, b_in, wdstT, wsrcT, wlinT, a1T, p1t,
             td_ref, ts_ref, xl_ref):
    h = _relu(_dot(x_ref[...], winT[...]) + b_in[...])
    q = _dot(pos_ref[...], p1t[...])
    td_ref[...] = jnp.concatenate(
        [_dot(_dot(h, wdstT[...]), a1T[...]), q], axis=1)
    ts_ref[...] = jnp.concatenate(
        [_dot(_dot(h, wsrcT[...]), a1T[...]), q], axis=1)
    xl_ref[...] = _dot(h, wlinT[...])


def _k1(x_pad, pos16, winT, b_in, wdstT, wsrcT, wlinT, a1T, p1t, npad):
    bn = 256
    grid = (npad // bn,)
    full = lambda shape: pl.BlockSpec(shape, lambda i: (0, 0))
    return pl.pallas_call(
        _k1_body,
        grid=grid,
        in_specs=[
            pl.BlockSpec((bn, 128), lambda i: (i, 0)),
            pl.BlockSpec((bn, 128), lambda i: (i, 0)),
            full((128, 128)), full((1, 128)),
            full((128, 128)), full((128, 128)), full((128, 128)),
            full((128, 64)), full((128, 64)),
        ],
        out_specs=[
            pl.BlockSpec((bn, 128), lambda i: (i, 0)),
            pl.BlockSpec((bn, 128), lambda i: (i, 0)),
            pl.BlockSpec((bn, 128), lambda i: (i, 0)),
        ],
        out_shape=[
            jax.ShapeDtypeStruct((npad, 128), F32),
            jax.ShapeDtypeStruct((npad, 128), F32),
            jax.ShapeDtypeStruct((npad, 128), F32),
        ],
    )(x_pad, pos16, winT, b_in, wdstT, wsrcT, wlinT, a1T, p1t)


def _k2_body(ep, src_hbm, dst_hbm, td_hbm, ts_hbm, xl_hbm,
             od, os_, ox,
             idxd0, idxs0, bd0, bs0, bx0,
             idxd1, idxs1, bd1, bs1, bx1,
             gsem0, gsem1, ssem0, ssem1):
    c = lax.axis_index("c")
    s = lax.axis_index("s")
    kt = ep // (NS * C)
    ka = ((kt * 34 + 99) // 100 // 2) * 2
    kb = kt - ka

    bufs = ((idxd0, idxs0, bd0, bs0, bx0, gsem0, ssem0),
            (idxd1, idxs1, bd1, bs1, bx1, gsem1, ssem1))

    def run(base0, nk):
        def fire_gather(k, p):
            idxd, idxs, bd, bs, bx, gsem, _ = bufs[p]
            pltpu.sync_copy(dst_hbm.at[pl.ds(base0 + k * C, C)], idxd)
            pltpu.sync_copy(src_hbm.at[pl.ds(base0 + k * C, C)], idxs)
            pltpu.async_copy(td_hbm.at[idxd], bd, gsem)
            pltpu.async_copy(ts_hbm.at[idxs], bs, gsem)
            pltpu.async_copy(xl_hbm.at[idxs], bx, gsem)

        def wait_gather(p):
            idxd, idxs, bd, bs, bx, gsem, _ = bufs[p]
            pltpu.make_async_copy(td_hbm.at[idxd], bd, gsem).wait()
            pltpu.make_async_copy(ts_hbm.at[idxs], bs, gsem).wait()
            pltpu.make_async_copy(xl_hbm.at[idxs], bx, gsem).wait()

        def fire_store(k, p):
            _, _, bd, bs, bx, _, ssem = bufs[p]
            sl = pl.ds(base0 + k * C, C)
            pltpu.async_copy(bd, od.at[sl], ssem)
            pltpu.async_copy(bs, os_.at[sl], ssem)
            pltpu.async_copy(bx, ox.at[sl], ssem)

        def wait_store(k, p):
            _, _, bd, bs, bx, _, ssem = bufs[p]
            sl = pl.ds(base0 + k * C, C)
            pltpu.make_async_copy(bd, od.at[sl], ssem).wait()
            pltpu.make_async_copy(bs, os_.at[sl], ssem).wait()
            pltpu.make_async_copy(bx, ox.at[sl], ssem).wait()

        fire_gather(0, 0)
        ng = nk // 2

        def body(g, _):
            k0 = 2 * g
            k1 = k0 + 1

            @pl.when(g > 0)
            def _():
                wait_store(k0 - 1, 1)

            fire_gather(k1, 1)
            wait_gather(0)
            fire_store(k0, 0)
            wait_store(k0, 0)

            @pl.when(g < ng - 1)
            def _():
                fire_gather(k0 + 2, 0)

            wait_gather(1)
            fire_store(k1, 1)
            return 0

        lax.fori_loop(0, ng, body, 0)
        wait_store(nk - 1, 1)

    @pl.when(c == 0)
    def _():
        run(s * ka * C, ka)

    @pl.when(c == 1)
    def _():
        run((NS * ka + s * kb) * C, kb)


def _k2(src, dst, td, ts, xl, ep):
    mesh = plsc.VectorSubcoreMesh(core_axis_name="c", subcore_axis_name="s")
    buf = lambda: [pltpu.VMEM((C,), jnp.int32), pltpu.VMEM((C,), jnp.int32),
                   pltpu.VMEM((C, 128), F32), pltpu.VMEM((C, 128), F32),
                   pltpu.VMEM((C, 128), F32)]
    kern = functools.partial(
        pl.kernel,
        compiler_params=pltpu.CompilerParams(use_tc_tiling_on_sc=False),
        out_type=[
            jax.ShapeDtypeStruct((ep, 128), F32),
            jax.ShapeDtypeStruct((ep, 128), F32),
            jax.ShapeDtypeStruct((ep, 128), F32),
        ],
        mesh=mesh,
        scratch_types=buf() + buf() + [pltpu.SemaphoreType.DMA] * 4,
    )(functools.partial(_k2_body, ep))
    return kern(src, dst, td, ts, xl)


def _k3_body(d_ref, s_ref, x_ref,
             e1sel, e2sel, pb1, p2t, pb2, a1t, ab1, a2t, ab2,
             ex_ref, pp_ref):
    diff = d_ref[...] - s_ref[...]
    d1 = _relu(_dot(diff, e2sel[...]) + pb1[...])
    delta = _relu(_dot(d1, p2t[...]) + pb2[...])
    z1 = _relu(_dot(delta, a1t[...]) + _dot(diff, e1sel[...])
               + ab1[...])
    alpha = _relu(_dot(z1, a2t[...]) + ab2[...])
    ex = jnp.exp(alpha)
    ex_ref[...] = ex
    pp_ref[...] = ex * (x_ref[...] + delta)


def _k3(dg, sg, xg, pb1, p2t, pb2, a1t, ab1, a2t, ab2, ep):
    be = 2048
    grid = (ep // be,)
    eye = jnp.eye(128, dtype=F32)
    e1sel = eye[:, :64]
    e2sel = eye[:, 64:]
    full = lambda shape: pl.BlockSpec(shape, lambda i: (0, 0))
    row = pl.BlockSpec((be, 128), lambda i: (i, 0))
    return pl.pallas_call(
        _k3_body,
        grid=grid,
        in_specs=[
            row, row, row,
            full((128, 64)), full((128, 64)),
            full((1, 64)), full((64, 128)), full((1, 128)),
            full((128, 64)), full((1, 64)), full((64, 128)), full((1, 128)),
        ],
        out_specs=[row, row],
        out_shape=[
            jax.ShapeDtypeStruct((ep, 128), F32),
            jax.ShapeDtypeStruct((ep, 128), F32),
        ],
    )(dg, sg, xg, e1sel, e2sel, pb1, p2t, pb2, a1t, ab1, a2t, ab2)


def _k4_body(ep, npad, dst_hbm, ex_hbm, pp_hbm, denom_hbm, num_hbm,
             acc, idxb0, rowb0, idxb1, rowb1, fsem0, fsem1):
    c = lax.axis_index("c")
    s = lax.axis_index("s")
    rows_per_tile = npad // NS
    r0 = s * rows_per_tile

    def zrow(j, _):
        for t in range(8):
            rowb0[j, pl.ds(t * 16, 16)] = jnp.zeros((16,), F32)
        return 0

    lax.fori_loop(0, C, zrow, 0)
    for t in range(rows_per_tile // C):
        pltpu.sync_copy(rowb0, acc.at[pl.ds(r0 + t * C, C)])
    plsc.subcore_barrier()

    et = ep // NS
    nk = et // C
    ng = nk // 2
    base0 = s * et
    bufs = ((idxb0, rowb0, fsem0), (idxb1, rowb1, fsem1))

    def chunk_from(src_arr):
        def fire(k, p):
            idxb, rowb, fsem = bufs[p]
            pltpu.sync_copy(dst_hbm.at[pl.ds(base0 + k * C, C)], idxb)
            pltpu.async_copy(src_arr.at[pl.ds(base0 + k * C, C)], rowb, fsem)

        def scat(k, p):
            idxb, rowb, fsem = bufs[p]
            pltpu.make_async_copy(
                src_arr.at[pl.ds(base0 + k * C, C)], rowb, fsem).wait()
            pltpu.sync_copy(rowb, acc.at[idxb], add=True)

        fire(0, 0)

        def body(g, _):
            k0 = 2 * g
            k1 = k0 + 1
            fire(k1, 1)
            scat(k0, 0)

            @pl.when(g < ng - 1)
            def _():
                fire(k0 + 2, 0)

            scat(k1, 1)
            return 0

        lax.fori_loop(0, ng, body, 0)

    @pl.when(c == 0)
    def _():
        chunk_from(ex_hbm)

    @pl.when(c == 1)
    def _():
        chunk_from(pp_hbm)

    plsc.subcore_barrier()
    for t in range(rows_per_tile // C):
        rows = pl.ds(r0 + t * C, C)

        @pl.when(c == 0)
        def _(rows=rows):
            pltpu.sync_copy(acc.at[rows], denom_hbm.at[rows])

        @pl.when(c == 1)
        def _(rows=rows):
            pltpu.sync_copy(acc.at[rows], num_hbm.at[rows])


def _k4(dst, ex, pp, ep, npad):
    mesh = plsc.VectorSubcoreMesh(core_axis_name="c", subcore_axis_name="s")
    kern = functools.partial(
        pl.kernel,
        out_type=[
            jax.ShapeDtypeStruct((npad, 128), F32),
            jax.ShapeDtypeStruct((npad, 128), F32),
        ],
        mesh=mesh,
        scratch_types=[
            pltpu.VMEM_SHARED((npad, 128), F32),
            pltpu.VMEM((C,), jnp.int32),
            pltpu.VMEM((C, 128), F32),
            pltpu.VMEM((C,), jnp.int32),
            pltpu.VMEM((C, 128), F32),
            pltpu.SemaphoreType.DMA,
            pltpu.SemaphoreType.DMA,
        ],
    )(functools.partial(_k4_body, ep, npad))
    return kern(dst, ex, pp)


def _k5_body(num_ref, den_ref, woutT, b_out, y_ref):
    out = num_ref[...] / (den_ref[...] + 1e-16)
    y_ref[...] = _relu(_dot(out, woutT[...]) + b_out[...])


def _k5(num, den, woutT, b_out, npad):
    bn = 256
    grid = (npad // bn,)
    full = lambda shape: pl.BlockSpec(shape, lambda i: (0, 0))
    return pl.pallas_call(
        _k5_body,
        grid=grid,
        in_specs=[
            pl.BlockSpec((bn, 128), lambda i: (i, 0)),
            pl.BlockSpec((bn, 128), lambda i: (i, 0)),
            full((128, 128)), full((1, 128)),
        ],
        out_specs=pl.BlockSpec((bn, 128), lambda i: (i, 0)),
        out_shape=jax.ShapeDtypeStruct((npad, 128), F32),
    )(num, den, woutT, b_out)


def kernel(x, pos, edge_index, Win, b_in, Wout, b_out, Wlin, Wsrc, Wdst,
           P1, pb1, P2, pb2, A1, ab1, A2, ab2):
    n, d = x.shape
    e = edge_index.shape[1]
    npad = ((n + 255) // 256) * 256
    ereal = e + n
    ep = ((ereal + 2 * NW * C - 1) // (2 * NW * C)) * (2 * NW * C)

    x_pad = jnp.pad(x, ((0, npad - n), (0, 0)))
    pos16 = jnp.pad(pos, ((0, npad - n), (0, 128 - pos.shape[1])))
    loop = jnp.arange(n, dtype=edge_index.dtype)
    pad_e = ep - ereal
    src = jnp.concatenate([edge_index[0], loop,
                           jnp.zeros((pad_e,), edge_index.dtype)])
    dst = jnp.concatenate([edge_index[1], loop,
                           jnp.full((pad_e,), n, edge_index.dtype)])

    winT = Win.T
    wdstT = Wdst.T
    wsrcT = Wsrc.T
    wlinT = Wlin.T
    woutT = Wout.T
    a1T = A1.T
    p1t = jnp.pad(P1.T, ((0, 128 - P1.shape[1]), (0, 0)))
    p2t = P2.T
    a2t = A2.T
    b_in2 = b_in.reshape(1, -1)
    b_out2 = b_out.reshape(1, -1)
    pb1_2 = pb1.reshape(1, -1)
    pb2_2 = pb2.reshape(1, -1)
    ab1_2 = ab1.reshape(1, -1)
    ab2_2 = ab2.reshape(1, -1)

    td, ts, xl = _k1(x_pad, pos16, winT, b_in2, wdstT, wsrcT, wlinT,
                     a1T, p1t, npad)
    dg, sg, xg = _k2(src, dst, td, ts, xl, ep)
    ex, pp = _k3(dg, sg, xg, pb1_2, p2t, pb2_2, a1T, ab1_2, a2t, ab2_2, ep)
    den, num = _k4(dst, ex, pp, ep, npad)
    y = _k5(num, den, woutT, b_out2, npad)
    return y[:n]


# speedup vs baseline: 6.7883x; 1.0337x over previous
import functools

import jax
import jax.numpy as jnp
from jax import lax
from jax.experimental import pallas as pl
from jax.experimental.pallas import tpu as pltpu
from jax.experimental.pallas import tpu_sc as plsc

NC = 2
NS = 16
NW = NC * NS
C = 128

F32 = jnp.float32


def _relu(v):
    return jnp.maximum(v, 0.0)


def _dot(a, b):
    return jax.lax.dot_general(a, b, (((1,), (0,)), ((), ())),
                               preferred_element_type=F32)


def _k1_body(x_ref, pos_ref, winT, b_in, wdstT, wsrcT, wlinT, a1T, p1t,
             td_ref, ts_ref, xl_ref):
    h = _relu(_dot(x_ref[...], winT[...]) + b_in[...])
    q = _dot(pos_ref[...], p1t[...])
    td_ref[...] = jnp.concatenate(
        [_dot(_dot(h, wdstT[...]), a1T[...]), q], axis=1)
    ts_ref[...] = jnp.concatenate(
        [_dot(_dot(h, wsrcT[...]), a1T[...]), q], axis=1)
    xl_ref[...] = _dot(h, wlinT[...])


def _k1(x_pad, pos16, winT, b_in, wdstT, wsrcT, wlinT, a1T, p1t, npad):
    bn = 256
    grid = (npad // bn,)
    full = lambda shape: pl.BlockSpec(shape, lambda i: (0, 0))
    return pl.pallas_call(
        _k1_body,
        grid=grid,
        in_specs=[
            pl.BlockSpec((bn, 128), lambda i: (i, 0)),
            pl.BlockSpec((bn, 128), lambda i: (i, 0)),
            full((128, 128)), full((1, 128)),
            full((128, 128)), full((128, 128)), full((128, 128)),
            full((128, 64)), full((128, 64)),
        ],
        out_specs=[
            pl.BlockSpec((bn, 128), lambda i: (i, 0)),
            pl.BlockSpec((bn, 128), lambda i: (i, 0)),
            pl.BlockSpec((bn, 128), lambda i: (i, 0)),
        ],
        out_shape=[
            jax.ShapeDtypeStruct((npad, 128), F32),
            jax.ShapeDtypeStruct((npad, 128), F32),
            jax.ShapeDtypeStruct((npad, 128), F32),
        ],
    )(x_pad, pos16, winT, b_in, wdstT, wsrcT, wlinT, a1T, p1t)


def _k2_body(ep, src_hbm, dst_hbm, td_hbm, ts_hbm, xl_hbm,
             od, os_, ox,
             idxd0, idxs0, bd0, bs0, bx0,
             idxd1, idxs1, bd1, bs1, bx1,
             gsem0, gsem1, ssem0, ssem1):
    c = lax.axis_index("c")
    s = lax.axis_index("s")
    kt = ep // (NS * C)
    ka = ((kt * 61 + 99) // 100 // 2) * 2
    kb = kt - ka

    bufs = ((idxd0, idxs0, bd0, bs0, bx0, gsem0, ssem0),
            (idxd1, idxs1, bd1, bs1, bx1, gsem1, ssem1))

    def run(base0, nk):
        def fire_gather(k, p):
            idxd, idxs, bd, bs, bx, gsem, _ = bufs[p]
            pltpu.sync_copy(dst_hbm.at[pl.ds(base0 + k * C, C)], idxd)
            pltpu.sync_copy(src_hbm.at[pl.ds(base0 + k * C, C)], idxs)
            pltpu.async_copy(td_hbm.at[idxd], bd, gsem)
            pltpu.async_copy(ts_hbm.at[idxs], bs, gsem)
            pltpu.async_copy(xl_hbm.at[idxs], bx, gsem)

        def wait_gather(p):
            idxd, idxs, bd, bs, bx, gsem, _ = bufs[p]
            pltpu.make_async_copy(td_hbm.at[idxd], bd, gsem).wait()
            pltpu.make_async_copy(ts_hbm.at[idxs], bs, gsem).wait()
            pltpu.make_async_copy(xl_hbm.at[idxs], bx, gsem).wait()

        def fire_store(k, p):
            _, _, bd, bs, bx, _, ssem = bufs[p]
            sl = pl.ds(base0 + k * C, C)
            pltpu.async_copy(bd, od.at[sl], ssem)
            pltpu.async_copy(bs, os_.at[sl], ssem)
            pltpu.async_copy(bx, ox.at[sl], ssem)

        def wait_store(k, p):
            _, _, bd, bs, bx, _, ssem = bufs[p]
            sl = pl.ds(base0 + k * C, C)
            pltpu.make_async_copy(bd, od.at[sl], ssem).wait()
            pltpu.make_async_copy(bs, os_.at[sl], ssem).wait()
            pltpu.make_async_copy(bx, ox.at[sl], ssem).wait()

        fire_gather(0, 0)
        ng = nk // 2

        def body(g, _):
            k0 = 2 * g
            k1 = k0 + 1

            @pl.when(g > 0)
            def _():
                wait_store(k0 - 1, 1)

            fire_gather(k1, 1)
            wait_gather(0)
            fire_store(k0, 0)
            wait_store(k0, 0)

            @pl.when(g < ng - 1)
            def _():
                fire_gather(k0 + 2, 0)

            wait_gather(1)
            fire_store(k1, 1)
            return 0

        lax.fori_loop(0, ng, body, 0)
        wait_store(nk - 1, 1)

    @pl.when(c == 0)
    def _():
        run(s * ka * C, ka)

    @pl.when(c == 1)
    def _():
        run((NS * ka + s * kb) * C, kb)


def _k2(src, dst, td, ts, xl, ep):
    mesh = plsc.VectorSubcoreMesh(core_axis_name="c", subcore_axis_name="s")
    buf = lambda: [pltpu.VMEM((C,), jnp.int32), pltpu.VMEM((C,), jnp.int32),
                   pltpu.VMEM((C, 128), F32), pltpu.VMEM((C, 128), F32),
                   pltpu.VMEM((C, 128), F32)]
    kern = functools.partial(
        pl.kernel,
        compiler_params=pltpu.CompilerParams(use_tc_tiling_on_sc=False),
        out_type=[
            jax.ShapeDtypeStruct((ep, 128), F32),
            jax.ShapeDtypeStruct((ep, 128), F32),
            jax.ShapeDtypeStruct((ep, 128), F32),
        ],
        mesh=mesh,
        scratch_types=buf() + buf() + [pltpu.SemaphoreType.DMA] * 4,
    )(functools.partial(_k2_body, ep))
    return kern(src, dst, td, ts, xl)


def _k3_body(d_ref, s_ref, x_ref,
             e1sel, e2sel, pb1, p2t, pb2, a1t, ab1, a2t, ab2,
             ex_ref, pp_ref):
    diff = d_ref[...] - s_ref[...]
    d1 = _relu(_dot(diff, e2sel[...]) + pb1[...])
    delta = _relu(_dot(d1, p2t[...]) + pb2[...])
    z1 = _relu(_dot(delta, a1t[...]) + _dot(diff, e1sel[...])
               + ab1[...])
    alpha = _relu(_dot(z1, a2t[...]) + ab2[...])
    ex = jnp.exp(alpha)
    ex_ref[...] = ex
    pp_ref[...] = ex * (x_ref[...] + delta)


def _k3(dg, sg, xg, pb1, p2t, pb2, a1t, ab1, a2t, ab2, ep):
    be = 2048
    grid = (ep // be,)
    eye = jnp.eye(128, dtype=F32)
    e1sel = eye[:, :64]
    e2sel = eye[:, 64:]
    full = lambda shape: pl.BlockSpec(shape, lambda i: (0, 0))
    row = pl.BlockSpec((be, 128), lambda i: (i, 0))
    return pl.pallas_call(
        _k3_body,
        grid=grid,
        in_specs=[
            row, row, row,
            full((128, 64)), full((128, 64)),
            full((1, 64)), full((64, 128)), full((1, 128)),
            full((128, 64)), full((1, 64)), full((64, 128)), full((1, 128)),
        ],
        out_specs=[row, row],
        out_shape=[
            jax.ShapeDtypeStruct((ep, 128), F32),
            jax.ShapeDtypeStruct((ep, 128), F32),
        ],
    )(dg, sg, xg, e1sel, e2sel, pb1, p2t, pb2, a1t, ab1, a2t, ab2)


def _k4_body(ep, npad, dst_hbm, ex_hbm, pp_hbm, denom_hbm, num_hbm,
             acc, idxb0, rowb0, idxb1, rowb1, fsem0, fsem1):
    c = lax.axis_index("c")
    s = lax.axis_index("s")
    rows_per_tile = npad // NS
    r0 = s * rows_per_tile

    def zrow(j, _):
        for t in range(8):
            rowb0[j, pl.ds(t * 16, 16)] = jnp.zeros((16,), F32)
        return 0

    lax.fori_loop(0, C, zrow, 0)
    for t in range(rows_per_tile // C):
        pltpu.sync_copy(rowb0, acc.at[pl.ds(r0 + t * C, C)])
    plsc.subcore_barrier()

    et = ep // NS
    nk = et // C
    ng = nk // 2
    base0 = s * et
    bufs = ((idxb0, rowb0, fsem0), (idxb1, rowb1, fsem1))

    def chunk_from(src_arr):
        def fire(k, p):
            idxb, rowb, fsem = bufs[p]
            pltpu.sync_copy(dst_hbm.at[pl.ds(base0 + k * C, C)], idxb)
            pltpu.async_copy(src_arr.at[pl.ds(base0 + k * C, C)], rowb, fsem)

        def scat(k, p):
            idxb, rowb, fsem = bufs[p]
            pltpu.make_async_copy(
                src_arr.at[pl.ds(base0 + k * C, C)], rowb, fsem).wait()
            pltpu.sync_copy(rowb, acc.at[idxb], add=True)

        fire(0, 0)

        def body(g, _):
            k0 = 2 * g
            k1 = k0 + 1
            fire(k1, 1)
            scat(k0, 0)

            @pl.when(g < ng - 1)
            def _():
                fire(k0 + 2, 0)

            scat(k1, 1)
            return 0

        lax.fori_loop(0, ng, body, 0)

    @pl.when(c == 0)
    def _():
        chunk_from(ex_hbm)

    @pl.when(c == 1)
    def _():
        chunk_from(pp_hbm)

    plsc.subcore_barrier()
    for t in range(rows_per_tile // C):
        rows = pl.ds(r0 + t * C, C)

        @pl.when(c == 0)
        def _(rows=rows):
            pltpu.sync_copy(acc.at[rows], denom_hbm.at[rows])

        @pl.when(c == 1)
        def _(rows=rows):
            pltpu.sync_copy(acc.at[rows], num_hbm.at[rows])


def _k4(dst, ex, pp, ep, npad):
    mesh = plsc.VectorSubcoreMesh(core_axis_name="c", subcore_axis_name="s")
    kern = functools.partial(
        pl.kernel,
        out_type=[
            jax.ShapeDtypeStruct((npad, 128), F32),
            jax.ShapeDtypeStruct((npad, 128), F32),
        ],
        mesh=mesh,
        scratch_types=[
            pltpu.VMEM_SHARED((npad, 128), F32),
            pltpu.VMEM((C,), jnp.int32),
            pltpu.VMEM((C, 128), F32),
            pltpu.VMEM((C,), jnp.int32),
            pltpu.VMEM((C, 128), F32),
            pltpu.SemaphoreType.DMA,
            pltpu.SemaphoreType.DMA,
        ],
    )(functools.partial(_k4_body, ep, npad))
    return kern(dst, ex, pp)


def _k5_body(num_ref, den_ref, woutT, b_out, y_ref):
    out = num_ref[...] / (den_ref[...] + 1e-16)
    y_ref[...] = _relu(_dot(out, woutT[...]) + b_out[...])


def _k5(num, den, woutT, b_out, npad):
    bn = 256
    grid = (npad // bn,)
    full = lambda shape: pl.BlockSpec(shape, lambda i: (0, 0))
    return pl.pallas_call(
        _k5_body,
        grid=grid,
        in_specs=[
            pl.BlockSpec((bn, 128), lambda i: (i, 0)),
            pl.BlockSpec((bn, 128), lambda i: (i, 0)),
            full((128, 128)), full((1, 128)),
        ],
        out_specs=pl.BlockSpec((bn, 128), lambda i: (i, 0)),
        out_shape=jax.ShapeDtypeStruct((npad, 128), F32),
    )(num, den, woutT, b_out)


def kernel(x, pos, edge_index, Win, b_in, Wout, b_out, Wlin, Wsrc, Wdst,
           P1, pb1, P2, pb2, A1, ab1, A2, ab2):
    n, d = x.shape
    e = edge_index.shape[1]
    npad = ((n + 255) // 256) * 256
    ereal = e + n
    ep = ((ereal + 2 * NW * C - 1) // (2 * NW * C)) * (2 * NW * C)

    x_pad = jnp.pad(x, ((0, npad - n), (0, 0)))
    pos16 = jnp.pad(pos, ((0, npad - n), (0, 128 - pos.shape[1])))
    loop = jnp.arange(n, dtype=edge_index.dtype)
    pad_e = ep - ereal
    src = jnp.concatenate([edge_index[0], loop,
                           jnp.zeros((pad_e,), edge_index.dtype)])
    dst = jnp.concatenate([edge_index[1], loop,
                           jnp.full((pad_e,), n, edge_index.dtype)])

    winT = Win.T
    wdstT = Wdst.T
    wsrcT = Wsrc.T
    wlinT = Wlin.T
    woutT = Wout.T
    a1T = A1.T
    p1t = jnp.pad(P1.T, ((0, 128 - P1.shape[1]), (0, 0)))
    p2t = P2.T
    a2t = A2.T
    b_in2 = b_in.reshape(1, -1)
    b_out2 = b_out.reshape(1, -1)
    pb1_2 = pb1.reshape(1, -1)
    pb2_2 = pb2.reshape(1, -1)
    ab1_2 = ab1.reshape(1, -1)
    ab2_2 = ab2.reshape(1, -1)

    td, ts, xl = _k1(x_pad, pos16, winT, b_in2, wdstT, wsrcT, wlinT,
                     a1T, p1t, npad)
    dg, sg, xg = _k2(src, dst, td, ts, xl, ep)
    ex, pp = _k3(dg, sg, xg, pb1_2, p2t, pb2_2, a1T, ab1_2, a2t, ab2_2, ep)
    den, num = _k4(dst, ex, pp, ep, npad)
    y = _k5(num, den, woutT, b_out2, npad)
    return y[:n]
